# baseline (speedup 1.0000x reference)
import functools

import jax
import jax.numpy as jnp
from jax import lax
from jax.experimental import pallas as pl
from jax.experimental.pallas import tpu as pltpu
from jax.experimental.pallas import tpu_sc as plsc

N = 10000
K = 16
D = 256
OUT = 512

NW = 32
PT = 320
NPAD = NW * PT
CH = 8
NCH = PT // CH
ROWS = CH * K
L = 16


def _sc_neigh_sum(x, neigh_chunked):
    mesh = plsc.VectorSubcoreMesh(core_axis_name="c", subcore_axis_name="s")

    @functools.partial(
        pl.kernel,
        out_type=jax.ShapeDtypeStruct((NPAD, D), jnp.float32),
        mesh=mesh,
        scratch_types=[
            pltpu.VMEM((NCH, ROWS), jnp.int32),
            pltpu.VMEM((2, ROWS, D), jnp.float32),
            pltpu.VMEM((CH, D), jnp.float32),
            pltpu.SemaphoreType.DMA,
            pltpu.SemaphoreType.DMA,
        ],
    )
    def sc_kernel(x_hbm, idx_hbm, out_hbm, idx_v, rows_v, acc_v, sem0, sem1):
        wid = lax.axis_index("s") * 2 + lax.axis_index("c")
        nbase = wid * PT
        sems = (sem0, sem1)

        pltpu.sync_copy(idx_hbm.at[wid], idx_v)

        def start_gather(cc, buf):
            return pltpu.async_copy(
                x_hbm.at[idx_v.at[cc]], rows_v.at[buf], sems[buf])

        def wait_gather(cc, buf):
            pltpu.make_async_copy(
                x_hbm.at[idx_v.at[cc]], rows_v.at[buf], sems[buf]).wait()

        start_gather(0, 0)
        start_gather(1, 1)

        @pl.loop(0, NCH, step=2)
        def _(c):
            for buf in range(2):
                cc = c + buf
                wait_gather(cc, buf)

                @pl.loop(0, CH)
                def _(n):
                    r0 = n * K

                    @pl.loop(0, D, step=L)
                    def _(dd):
                        acc = rows_v[buf, r0, pl.ds(dd, L)]
                        for k in range(1, K):
                            acc = acc + rows_v[buf, r0 + k, pl.ds(dd, L)]
                        acc_v[n, pl.ds(dd, L)] = acc

                pltpu.sync_copy(acc_v, out_hbm.at[pl.ds(nbase + cc * CH, CH)])

                @pl.when(cc + 2 < NCH)
                def _():
                    start_gather(cc + 2, buf)

    return sc_kernel(x, neigh_chunked)


BN = 1000


def _tc_body(x_ref, s_ref, w_ref, b_ref, o_ref):
    acc = lax.dot_general(
        x_ref[...], w_ref[:, :D], (((1,), (1,)), ((), ())),
        preferred_element_type=jnp.float32,
        precision=lax.Precision.HIGHEST)
    acc2 = lax.dot_general(
        s_ref[...], w_ref[:, D:], (((1,), (1,)), ((), ())),
        preferred_element_type=jnp.float32,
        precision=lax.Precision.HIGHEST)
    acc = acc + acc2 * (1.0 / K) + b_ref[...]
    o_ref[...] = jnp.maximum(acc, 0.0)


def _tc_linear(x, s, W, b2d):
    return pl.pallas_call(
        _tc_body,
        grid=(N // BN,),
        in_specs=[
            pl.BlockSpec((BN, D), lambda i: (i, 0)),
            pl.BlockSpec((BN, D), lambda i: (i, 0)),
            pl.BlockSpec((OUT, 2 * D), lambda i: (0, 0)),
            pl.BlockSpec((1, OUT), lambda i: (0, 0)),
        ],
        out_specs=pl.BlockSpec((BN, OUT), lambda i: (i, 0)),
        out_shape=jax.ShapeDtypeStruct((N, OUT), jnp.float32),
    )(x, s, W, b2d)


def kernel(x, neigh, W, b):
    neigh_pad = jnp.pad(neigh, ((0, NPAD - N), (0, 0)))
    s = _sc_neigh_sum(x, neigh_pad.reshape(NW, NCH, ROWS))
    return _tc_linear(x, s, W, b.reshape(1, OUT))


# speedup vs baseline: 1.0036x; 1.0036x over previous
import functools

import jax
import jax.numpy as jnp
from jax import lax
from jax.experimental import pallas as pl
from jax.experimental.pallas import tpu as pltpu
from jax.experimental.pallas import tpu_sc as plsc

N = 10000
K = 16
D = 256
OUT = 512

NW = 32
PT = 320
NPAD = NW * PT
CH = 4
NCH = PT // CH
ROWS = CH * K
NBUF = 4
L = 16


def _sc_neigh_sum(x, neigh_chunked):
    mesh = plsc.VectorSubcoreMesh(core_axis_name="c", subcore_axis_name="s")

    @functools.partial(
        pl.kernel,
        out_type=jax.ShapeDtypeStruct((NPAD, D), jnp.float32),
        mesh=mesh,
        scratch_types=[
            pltpu.VMEM((NCH, ROWS), jnp.int32),
            pltpu.VMEM((NBUF, ROWS, D), jnp.float32),
            pltpu.VMEM((CH, D), jnp.float32),
        ] + [pltpu.SemaphoreType.DMA] * NBUF,
    )
    def sc_kernel(x_hbm, idx_hbm, out_hbm, idx_v, rows_v, acc_v, *sems):
        wid = lax.axis_index("s") * 2 + lax.axis_index("c")
        nbase = wid * PT

        pltpu.sync_copy(idx_hbm.at[wid], idx_v)

        def start_gather(cc, buf):
            return pltpu.async_copy(
                x_hbm.at[idx_v.at[cc]], rows_v.at[buf], sems[buf])

        def wait_gather(cc, buf):
            pltpu.make_async_copy(
                x_hbm.at[idx_v.at[cc]], rows_v.at[buf], sems[buf]).wait()

        for b in range(NBUF):
            start_gather(b, b)

        @pl.loop(0, NCH, step=NBUF)
        def _(c):
            for buf in range(NBUF):
                cc = c + buf
                wait_gather(cc, buf)

                @pl.loop(0, CH)
                def _(n):
                    r0 = n * K

                    @pl.loop(0, D, step=L)
                    def _(dd):
                        acc = rows_v[buf, r0, pl.ds(dd, L)]
                        for k in range(1, K):
                            acc = acc + rows_v[buf, r0 + k, pl.ds(dd, L)]
                        acc_v[n, pl.ds(dd, L)] = acc

                pltpu.sync_copy(acc_v, out_hbm.at[pl.ds(nbase + cc * CH, CH)])

                @pl.when(cc + NBUF < NCH)
                def _():
                    start_gather(cc + NBUF, buf)

    return sc_kernel(x, neigh_chunked)


BN = 1000


def _tc_body(x_ref, s_ref, w_ref, b_ref, o_ref):
    acc = lax.dot_general(
        x_ref[...], w_ref[:, :D], (((1,), (1,)), ((), ())),
        preferred_element_type=jnp.float32,
        precision=lax.Precision.HIGHEST)
    acc2 = lax.dot_general(
        s_ref[...], w_ref[:, D:], (((1,), (1,)), ((), ())),
        preferred_element_type=jnp.float32,
        precision=lax.Precision.HIGHEST)
    acc = acc + acc2 * (1.0 / K) + b_ref[...]
    o_ref[...] = jnp.maximum(acc, 0.0)


def _tc_linear(x, s, W, b2d):
    return pl.pallas_call(
        _tc_body,
        grid=(N // BN,),
        in_specs=[
            pl.BlockSpec((BN, D), lambda i: (i, 0)),
            pl.BlockSpec((BN, D), lambda i: (i, 0)),
            pl.BlockSpec((OUT, 2 * D), lambda i: (0, 0)),
            pl.BlockSpec((1, OUT), lambda i: (0, 0)),
        ],
        out_specs=pl.BlockSpec((BN, OUT), lambda i: (i, 0)),
        out_shape=jax.ShapeDtypeStruct((N, OUT), jnp.float32),
    )(x, s, W, b2d)


def kernel(x, neigh, W, b):
    neigh_pad = jnp.pad(neigh, ((0, NPAD - N), (0, 0)))
    s = _sc_neigh_sum(x, neigh_pad.reshape(NW, NCH, ROWS))
    return _tc_linear(x, s, W, b.reshape(1, OUT))
